# baseline (device time: 88638 ns/iter reference)
import jax
import jax.numpy as jnp
from jax import lax
from jax.experimental import pallas as pl
from jax.experimental.pallas import tpu as pltpu

N_DEV = 32
SQ = 512
D = 1024
HQ = 8
DH = 128
SKV = 2048
ROWS = SQ // N_DEV
SCALE = 0.08838834764831843


def kernel(x, Wq, Wo, K_ext, V_ext):
    B_BLK = 128
    CPB = B_BLK // ROWS
    N_GRP = SQ // B_BLK

    def body(x_ref, wq_ref, wo_ref, k_ref, v_ref, out_ref,
             xcat_ref, stage_ref, rs_buf, red_ref,
             rs_send, rs_recv, ag_send, ag_recv):
        my = lax.axis_index("i")

        bf16 = jnp.bfloat16
        xv = x_ref[0]
        xcat_ref[:SQ] = xv
        xcat_ref[SQ:] = xv[:B_BLK]
        wq16 = wq_ref[...].astype(bf16)
        wo16 = wo_ref[...].astype(bf16)
        kv = k_ref[0].reshape(SKV, HQ * DH).astype(bf16)
        vv = v_ref[0].reshape(SKV, HQ * DH).astype(bf16)
        rs_descs = []
        for g in range(N_GRP):
            row0 = lax.rem(my + 1 + CPB * g, N_DEV) * ROWS
            xb = xcat_ref[pl.ds(row0, B_BLK), :].astype(bf16)
            qb = jnp.dot(xb, wq16, preferred_element_type=jnp.float32)
            qb = qb.astype(bf16)
            outs = []
            for h in range(HQ):
                qh = qb[:, h * DH:(h + 1) * DH]
                kh = kv[:, h * DH:(h + 1) * DH]
                vh = vv[:, h * DH:(h + 1) * DH]
                s = lax.dot_general(
                    qh, kh, (((1,), (1,)), ((), ())),
                    preferred_element_type=jnp.float32,
                ) * SCALE
                m = jnp.max(s, axis=1, keepdims=True)
                p = jnp.exp(s - m)
                l = jnp.sum(p, axis=1, keepdims=True)
                o = jnp.dot(p.astype(bf16), vh,
                            preferred_element_type=jnp.float32) / l
                outs.append(o)
            attnb = jnp.concatenate(outs, axis=1).astype(bf16)
            pb = jnp.dot(attnb, wo16, preferred_element_type=jnp.float32)
            stage_ref[g] = pb.reshape(CPB, ROWS, D)
            for j in range(CPB):
                off = 1 + CPB * g + j
                if off == N_DEV:
                    continue
                tgt = lax.rem(my + off, N_DEV)
                d = pltpu.make_async_remote_copy(
                    src_ref=stage_ref.at[g, j],
                    dst_ref=rs_buf.at[off],
                    send_sem=rs_send.at[off],
                    recv_sem=rs_recv.at[off],
                    device_id=(tgt,),
                    device_id_type=pl.DeviceIdType.MESH,
                )
                d.start()
                rs_descs.append(d)

        rs_buf[0] = stage_ref[N_GRP - 1, CPB - 1]

        for d in rs_descs:
            d.wait_recv()
        red = jnp.sum(rs_buf[...], axis=0)
        red_ref[...] = red

        out_ref[0, pl.ds(my * ROWS, ROWS), :] = red
        ag_descs = []
        for off in range(1, N_DEV):
            tgt = lax.rem(my + off, N_DEV)
            d = pltpu.make_async_remote_copy(
                src_ref=red_ref,
                dst_ref=out_ref.at[0, pl.ds(my * ROWS, ROWS), :],
                send_sem=ag_send.at[off],
                recv_sem=ag_recv.at[off],
                device_id=(tgt,),
                device_id_type=pl.DeviceIdType.MESH,
            )
            d.start()
            ag_descs.append(d)

        for d in ag_descs:
            d.wait_recv()
        for d in rs_descs:
            d.wait_send()
        for d in ag_descs:
            d.wait_send()

    return pl.pallas_call(
        body,
        out_shape=jax.ShapeDtypeStruct((1, SQ, D), jnp.float32),
        in_specs=[pl.BlockSpec(memory_space=pltpu.VMEM)] * 5,
        out_specs=pl.BlockSpec(memory_space=pltpu.VMEM),
        scratch_shapes=[
            pltpu.VMEM((SQ + 128, D), jnp.float32),
            pltpu.VMEM((SQ // 128, 128 // ROWS, ROWS, D), jnp.float32),
            pltpu.VMEM((N_DEV, ROWS, D), jnp.float32),
            pltpu.VMEM((ROWS, D), jnp.float32),
            pltpu.SemaphoreType.DMA((N_DEV,)),
            pltpu.SemaphoreType.DMA((N_DEV,)),
            pltpu.SemaphoreType.DMA((N_DEV,)),
            pltpu.SemaphoreType.DMA((N_DEV,)),
        ],
    )(x, Wq, Wo, K_ext, V_ext)


# device time: 79473 ns/iter; 1.1153x vs baseline; 1.1153x over previous
import os

import jax
import jax.numpy as jnp
from jax import lax
from jax.experimental import pallas as pl
from jax.experimental.pallas import tpu as pltpu

_ABLATE = os.environ.get("ABLATE", "")

N_DEV = 32
SQ = 512
D = 1024
HQ = 8
DH = 128
SKV = 2048
ROWS = SQ // N_DEV
SCALE = 0.08838834764831843


def kernel(x, Wq, Wo, K_ext, V_ext):
    B_BLK = 128
    CPB = B_BLK // ROWS
    N_GRP = SQ // B_BLK

    def body(x_ref, wq_ref, wo_ref, k_ref, v_ref, out_ref,
             xcat_ref, stage_ref, rs_buf, red_ref,
             rs_send, rs_recv, ag_send, ag_recv):
        my = lax.axis_index("i")

        bf16 = jnp.bfloat16
        xv = x_ref[0]
        xcat_ref[:SQ] = xv
        xcat_ref[SQ:] = xv[:B_BLK]
        wq16 = wq_ref[...].astype(bf16)
        wo16 = wo_ref[...].astype(bf16)
        kv = k_ref[0].reshape(SKV, HQ * DH).astype(bf16)
        vv = v_ref[0].reshape(SKV, HQ * DH).astype(bf16)
        rs_descs = []
        for g in range(N_GRP):
            row0 = lax.rem(my + 1 + CPB * g, N_DEV) * ROWS
            xb = xcat_ref[pl.ds(row0, B_BLK), :].astype(bf16)
            if _ABLATE == "comm":
                stage_ref[g] = xcat_ref[pl.ds(row0, B_BLK), :].reshape(
                    CPB, ROWS, D)
                for j in range(CPB):
                    off = 1 + CPB * g + j
                    if off == N_DEV:
                        continue
                    tgt = lax.rem(my + off, N_DEV)
                    d = pltpu.make_async_remote_copy(
                        src_ref=stage_ref.at[g, j],
                        dst_ref=rs_buf.at[off],
                        send_sem=rs_send.at[off],
                        recv_sem=rs_recv.at[off],
                        device_id=(tgt,),
                        device_id_type=pl.DeviceIdType.MESH,
                    )
                    d.start()
                    rs_descs.append(d)
                continue
            qb = jnp.dot(xb, wq16, preferred_element_type=jnp.float32)
            qb = qb.astype(bf16)
            outs = []
            for h in range(HQ):
                qh = qb[:, h * DH:(h + 1) * DH]
                kh = kv[:, h * DH:(h + 1) * DH]
                vh = vv[:, h * DH:(h + 1) * DH]
                s = lax.dot_general(
                    qh, kh, (((1,), (1,)), ((), ())),
                    preferred_element_type=jnp.float32,
                ) * SCALE
                m = jnp.max(s, axis=1, keepdims=True)
                p = jnp.exp(s - m)
                l = jnp.sum(p, axis=1, keepdims=True)
                o = jnp.dot(p.astype(bf16), vh,
                            preferred_element_type=jnp.float32) / l
                outs.append(o)
            attnb = jnp.concatenate(outs, axis=1).astype(bf16)
            pb = jnp.dot(attnb, wo16, preferred_element_type=jnp.float32)
            stage_ref[g] = pb.reshape(CPB, ROWS, D)
            for j in range(CPB):
                off = 1 + CPB * g + j
                if off == N_DEV or _ABLATE == "compute":
                    continue
                tgt = lax.rem(my + off, N_DEV)
                d = pltpu.make_async_remote_copy(
                    src_ref=stage_ref.at[g, j],
                    dst_ref=rs_buf.at[off],
                    send_sem=rs_send.at[off],
                    recv_sem=rs_recv.at[off],
                    device_id=(tgt,),
                    device_id_type=pl.DeviceIdType.MESH,
                )
                d.start()
                rs_descs.append(d)

        rs_buf[0] = stage_ref[N_GRP - 1, CPB - 1]

        for d in rs_descs:
            d.wait_recv()
        red = jnp.sum(rs_buf[...], axis=0)
        red_ref[...] = red

        out_ref[0, pl.ds(my * ROWS, ROWS), :] = red
        ag_descs = []
        for off in range(1, N_DEV if _ABLATE != "compute" else 1):
            tgt = lax.rem(my + off, N_DEV)
            d = pltpu.make_async_remote_copy(
                src_ref=red_ref,
                dst_ref=out_ref.at[0, pl.ds(my * ROWS, ROWS), :],
                send_sem=ag_send.at[off],
                recv_sem=ag_recv.at[off],
                device_id=(tgt,),
                device_id_type=pl.DeviceIdType.MESH,
            )
            d.start()
            ag_descs.append(d)

        for d in ag_descs:
            d.wait_recv()
        for d in rs_descs:
            d.wait_send()
        for d in ag_descs:
            d.wait_send()

    return pl.pallas_call(
        body,
        out_shape=jax.ShapeDtypeStruct((1, SQ, D), jnp.float32),
        in_specs=[pl.BlockSpec(memory_space=pltpu.VMEM)] * 5,
        out_specs=pl.BlockSpec(memory_space=pltpu.VMEM),
        scratch_shapes=[
            pltpu.VMEM((SQ + 128, D), jnp.float32),
            pltpu.VMEM((SQ // 128, 128 // ROWS, ROWS, D), jnp.float32),
            pltpu.VMEM((N_DEV, ROWS, D), jnp.float32),
            pltpu.VMEM((ROWS, D), jnp.float32),
            pltpu.SemaphoreType.DMA((N_DEV,)),
            pltpu.SemaphoreType.DMA((N_DEV,)),
            pltpu.SemaphoreType.DMA((N_DEV,)),
            pltpu.SemaphoreType.DMA((N_DEV,)),
        ],
    )(x, Wq, Wo, K_ext, V_ext)


# device time: 70032 ns/iter; 1.2657x vs baseline; 1.1348x over previous
import os

import jax
import jax.numpy as jnp
from jax import lax
from jax.experimental import pallas as pl
from jax.experimental.pallas import tpu as pltpu

_ABLATE = os.environ.get("ABLATE", "")

N_DEV = 32
SQ = 512
D = 1024
HQ = 8
DH = 128
SKV = 2048
ROWS = SQ // N_DEV
SCALE = 0.08838834764831843


def kernel(x, Wq, Wo, K_ext, V_ext):
    B_BLK = 128
    CPB = B_BLK // ROWS
    N_GRP = SQ // B_BLK

    def body(x_ref, wq_ref, wo_ref, k_ref, v_ref, out_ref,
             xcat_ref, stage_ref, rs_buf, red_ref, ag_buf,
             rs_send, rs_recv, ag_send, ag_recv):
        my = lax.axis_index("i")

        bf16 = jnp.bfloat16
        xv = x_ref[0]
        xcat_ref[:SQ] = xv
        xcat_ref[SQ:] = xv[:B_BLK]
        wq16 = wq_ref[...].astype(bf16)
        wo16 = wo_ref[...].astype(bf16)
        kv = k_ref[0].reshape(SKV, HQ * DH).astype(bf16)
        vv = v_ref[0].reshape(SKV, HQ * DH).astype(bf16)
        rs_descs = []
        for g in range(N_GRP):
            row0 = lax.rem(my + 1 + CPB * g, N_DEV) * ROWS
            xb = xcat_ref[pl.ds(row0, B_BLK), :].astype(bf16)
            if _ABLATE == "comm":
                stage_ref[g] = xcat_ref[pl.ds(row0, B_BLK), :].astype(
                    bf16).reshape(CPB, ROWS, D)
                for j in range(CPB):
                    off = 1 + CPB * g + j
                    if off == N_DEV:
                        continue
                    tgt = lax.rem(my + off, N_DEV)
                    d = pltpu.make_async_remote_copy(
                        src_ref=stage_ref.at[g, j],
                        dst_ref=rs_buf.at[off],
                        send_sem=rs_send.at[off],
                        recv_sem=rs_recv.at[off],
                        device_id=(tgt,),
                        device_id_type=pl.DeviceIdType.MESH,
                    )
                    d.start()
                    rs_descs.append(d)
                continue
            qb = jnp.dot(xb, wq16, preferred_element_type=jnp.float32)
            qb = qb.astype(bf16)
            outs = []
            for h in range(HQ):
                qh = qb[:, h * DH:(h + 1) * DH]
                kh = kv[:, h * DH:(h + 1) * DH]
                vh = vv[:, h * DH:(h + 1) * DH]
                s = lax.dot_general(
                    qh, kh, (((1,), (1,)), ((), ())),
                    preferred_element_type=jnp.float32,
                ) * SCALE
                m = jnp.max(s, axis=1, keepdims=True)
                p = jnp.exp(s - m)
                l = jnp.sum(p, axis=1, keepdims=True)
                o = jnp.dot(p.astype(bf16), vh,
                            preferred_element_type=jnp.float32) / l
                outs.append(o)
            attnb = jnp.concatenate(outs, axis=1).astype(bf16)
            pb = jnp.dot(attnb, wo16, preferred_element_type=jnp.float32)
            stage_ref[g] = pb.astype(bf16).reshape(CPB, ROWS, D)
            for j in range(CPB):
                off = 1 + CPB * g + j
                if off == N_DEV or _ABLATE == "compute":
                    continue
                tgt = lax.rem(my + off, N_DEV)
                d = pltpu.make_async_remote_copy(
                    src_ref=stage_ref.at[g, j],
                    dst_ref=rs_buf.at[off],
                    send_sem=rs_send.at[off],
                    recv_sem=rs_recv.at[off],
                    device_id=(tgt,),
                    device_id_type=pl.DeviceIdType.MESH,
                )
                d.start()
                rs_descs.append(d)

        rs_buf[0] = stage_ref[N_GRP - 1, CPB - 1]

        for d in rs_descs:
            d.wait_recv()
        red = jnp.sum(rs_buf[...].astype(jnp.float32), axis=0)
        red_ref[...] = red.astype(bf16)

        out_ref[0, pl.ds(my * ROWS, ROWS), :] = red
        ag_descs = []
        for off in range(1, N_DEV if _ABLATE != "compute" else 1):
            tgt = lax.rem(my + off, N_DEV)
            d = pltpu.make_async_remote_copy(
                src_ref=red_ref,
                dst_ref=ag_buf.at[off],
                send_sem=ag_send.at[off],
                recv_sem=ag_recv.at[off],
                device_id=(tgt,),
                device_id_type=pl.DeviceIdType.MESH,
            )
            d.start()
            ag_descs.append(d)

        for d in ag_descs:
            d.wait_recv()
        for off in range(1, N_DEV if _ABLATE != "compute" else 1):
            src_dev = lax.rem(my - off + N_DEV, N_DEV)
            out_ref[0, pl.ds(src_dev * ROWS, ROWS), :] = (
                ag_buf[off].astype(jnp.float32))
        for d in rs_descs:
            d.wait_send()
        for d in ag_descs:
            d.wait_send()

    return pl.pallas_call(
        body,
        out_shape=jax.ShapeDtypeStruct((1, SQ, D), jnp.float32),
        in_specs=[pl.BlockSpec(memory_space=pltpu.VMEM)] * 5,
        out_specs=pl.BlockSpec(memory_space=pltpu.VMEM),
        scratch_shapes=[
            pltpu.VMEM((SQ + 128, D), jnp.float32),
            pltpu.VMEM((SQ // 128, 128 // ROWS, ROWS, D), jnp.bfloat16),
            pltpu.VMEM((N_DEV, ROWS, D), jnp.bfloat16),
            pltpu.VMEM((ROWS, D), jnp.bfloat16),
            pltpu.VMEM((N_DEV, ROWS, D), jnp.bfloat16),
            pltpu.SemaphoreType.DMA((N_DEV,)),
            pltpu.SemaphoreType.DMA((N_DEV,)),
            pltpu.SemaphoreType.DMA((N_DEV,)),
            pltpu.SemaphoreType.DMA((N_DEV,)),
        ],
    )(x, Wq, Wo, K_ext, V_ext)
